# baseline (speedup 1.0000x reference)
import functools

import jax
import jax.numpy as jnp
from jax.experimental import pallas as pl


def _add_emb_kernel(x_ref, table_ref, o_ref, *, F):
    t = table_ref[0:F, :]
    norm = jnp.sqrt(jnp.sum(t * t, axis=-1, keepdims=True))
    scale = jnp.where(norm > 1.0, 1.0 / jnp.maximum(norm, 1e-12), 1.0)
    t = t * scale
    row = jax.lax.broadcasted_iota(jnp.int32, (F, F), 0)
    col = jax.lax.broadcasted_iota(jnp.int32, (F, F), 1)
    perm = (row + col == F - 1).astype(jnp.float32)
    emb = jnp.dot(perm, t, preferred_element_type=jnp.float32)
    o_ref[...] = x_ref[...] + emb[None, :, None, :]


def kernel(x, in_F, out_F, table):
    B, F, N, D = x.shape
    BB = 8
    return pl.pallas_call(
        functools.partial(_add_emb_kernel, F=F),
        grid=(B // BB,),
        in_specs=[
            pl.BlockSpec((BB, F, N, D), lambda i: (i, 0, 0, 0)),
            pl.BlockSpec((table.shape[0], D), lambda i: (0, 0)),
        ],
        out_specs=pl.BlockSpec((BB, F, N, D), lambda i: (i, 0, 0, 0)),
        out_shape=jax.ShapeDtypeStruct(x.shape, x.dtype),
    )(x, table)


# speedup vs baseline: 1.1814x; 1.1814x over previous
import functools

import jax
import jax.numpy as jnp
from jax.experimental import pallas as pl
from jax.experimental.pallas import tpu as pltpu


def _add_emb_kernel(x_ref, table_ref, o_ref, emb_ref, *, F, N):
    @pl.when(pl.program_id(0) == 0)
    def _compute_emb():
        t = table_ref[0:F, :]
        norm = jnp.sqrt(jnp.sum(t * t, axis=-1, keepdims=True))
        scale = jnp.where(norm > 1.0, 1.0 / jnp.maximum(norm, 1e-12), 1.0)
        t = t * scale
        r = jax.lax.broadcasted_iota(jnp.int32, (F * N, F), 0)
        c = jax.lax.broadcasted_iota(jnp.int32, (F * N, F), 1)
        sel = (c == (F - 1 - r // N)).astype(jnp.float32)
        emb_ref[...] = jnp.dot(sel, t, preferred_element_type=jnp.float32)

    o_ref[...] = x_ref[...] + emb_ref[...][None, :, :]


def kernel(x, in_F, out_F, table):
    B, F, N, D = x.shape
    BB = 8
    xv = x.reshape(B, F * N, D)
    out = pl.pallas_call(
        functools.partial(_add_emb_kernel, F=F, N=N),
        grid=(B // BB,),
        in_specs=[
            pl.BlockSpec((BB, F * N, D), lambda i: (i, 0, 0)),
            pl.BlockSpec((table.shape[0], D), lambda i: (0, 0)),
        ],
        out_specs=pl.BlockSpec((BB, F * N, D), lambda i: (i, 0, 0)),
        out_shape=jax.ShapeDtypeStruct((B, F * N, D), x.dtype),
        scratch_shapes=[pltpu.VMEM((F * N, D), jnp.float32)],
    )(xv, table)
    return out.reshape(B, F, N, D)
